# Initial kernel scaffold; baseline (speedup 1.0000x reference)
import jax
import jax.numpy as jnp
from jax.experimental import pallas as pl


def kernel(feats, edge_index, W_src, W_dst, attn, bias):
    raise NotImplementedError("write your pallas kernel here")


# speedup vs baseline: 12.3570x; 12.3570x over previous
import functools

import jax
import jax.numpy as jnp
from jax import lax
from jax.experimental import pallas as pl
from jax.experimental.pallas import tpu as pltpu
from jax.experimental.pallas import tpu_sc as plsc

N = 10000
E = 160000
IN_FEATS = 256
OUT_FEATS = 64
HEADS = 8
F = HEADS * OUT_FEATS
NEG_SLOPE = 0.2

NC = 2
NS = 16
NW = NC * NS

_MB = 1000


def _mm_body(x_ref, ws_ref, wd_ref, fs_ref, fd_ref):
    x = x_ref[...]
    fs_ref[...] = jnp.dot(x, ws_ref[...], preferred_element_type=jnp.float32)
    fd_ref[...] = jnp.dot(x, wd_ref[...], preferred_element_type=jnp.float32)


_mm = pl.pallas_call(
    _mm_body,
    grid=(N // _MB,),
    in_specs=[
        pl.BlockSpec((_MB, IN_FEATS), lambda i: (i, 0)),
        pl.BlockSpec((IN_FEATS, F), lambda i: (0, 0)),
        pl.BlockSpec((IN_FEATS, F), lambda i: (0, 0)),
    ],
    out_specs=[
        pl.BlockSpec((_MB, F), lambda i: (i, 0)),
        pl.BlockSpec((_MB, F), lambda i: (i, 0)),
    ],
    out_shape=[jax.ShapeDtypeStruct((N, F), jnp.float32)] * 2,
)

_GB = 40
_EPW = E // NW

@functools.cache
def _sc_mesh():
    return plsc.VectorSubcoreMesh(
        core_axis_name="c", subcore_axis_name="s", num_cores=NC, num_subcores=NS)


def _gather_body(fs_hbm, fd_hbm, src_hbm, dst_hbm, ga_hbm, gb_hbm,
                 si, di, abuf, bbuf, sem_a, sem_b):
    wid = lax.axis_index("s") * NC + lax.axis_index("c")
    base = wid * _EPW

    def step(i, carry):
        off = base + i * _GB
        pltpu.sync_copy(src_hbm.at[pl.ds(off, _GB)], si)
        pltpu.sync_copy(dst_hbm.at[pl.ds(off, _GB)], di)
        cp_a = pltpu.async_copy(fs_hbm.at[si], abuf, sem_a)
        cp_b = pltpu.async_copy(fd_hbm.at[di], bbuf, sem_b)
        cp_a.wait()
        cp_b.wait()
        pltpu.sync_copy(abuf, ga_hbm.at[pl.ds(off, _GB)])
        pltpu.sync_copy(bbuf, gb_hbm.at[pl.ds(off, _GB)])
        return carry

    lax.fori_loop(0, _EPW // _GB, step, 0)


@functools.cache
def _gather():
    return pl.kernel(
        _gather_body,
        out_type=[jax.ShapeDtypeStruct((E, F), jnp.float32),
                  jax.ShapeDtypeStruct((E, F), jnp.float32)],
        mesh=_sc_mesh(),
        scratch_types=[
            pltpu.VMEM((_GB,), jnp.int32),
            pltpu.VMEM((_GB,), jnp.int32),
            pltpu.VMEM((_GB, F), jnp.float32),
            pltpu.VMEM((_GB, F), jnp.float32),
            pltpu.SemaphoreType.DMA,
            pltpu.SemaphoreType.DMA,
        ],
    )

_EB = 2000


def _edge_body(ga_ref, gb_ref, a_ref, x_ref, msg_ref):
    ga = ga_ref[...]
    t = ga + gb_ref[...]
    t = jnp.maximum(t, NEG_SLOPE * t)
    ex = jnp.exp(jnp.dot(t, a_ref[...], preferred_element_type=jnp.float32))
    msg = ga * jnp.dot(ex, x_ref[...], preferred_element_type=jnp.float32)
    for c in range(4):
        msg_ref[c] = msg[:, c * 128:(c + 1) * 128]
    msg_ref[4] = jnp.concatenate(
        [ex, jnp.zeros((_EB, 112), jnp.float32)], axis=1)


_edge = pl.pallas_call(
    _edge_body,
    grid=(E // _EB,),
    in_specs=[
        pl.BlockSpec((_EB, F), lambda i: (i, 0)),
        pl.BlockSpec((_EB, F), lambda i: (i, 0)),
        pl.BlockSpec((F, 16), lambda i: (0, 0)),
        pl.BlockSpec((16, F), lambda i: (0, 0)),
    ],
    out_specs=pl.BlockSpec((5, _EB, 128), lambda i: (0, i, 0)),
    out_shape=jax.ShapeDtypeStruct((5, E, 128), jnp.float32),
)

_SB = 80
_EPT = E // NS
_ZR = 624
_ZTAIL = N - NS * _ZR


_ZB = 104


_SB4 = 40


def _scatter_body(msgf_hbm, dst_hbm, z128_hbm,
                  unf_hbm, acc, vbuf, idxb, zbuf, vbuf4, idxb4):
    cid = lax.axis_index("c")
    sid = lax.axis_index("s")
    nbase = sid * _ZR

    pltpu.sync_copy(z128_hbm.at[pl.ds(0, _ZB)], zbuf)

    for rep in range(2):
        chunk = cid * 2 + rep
        for j in range(_ZR // _ZB):
            pltpu.sync_copy(zbuf, acc.at[pl.ds(nbase + j * _ZB, _ZB)])

        @pl.when(sid == NS - 1)
        def _():
            pltpu.sync_copy(zbuf.at[pl.ds(0, _ZTAIL)],
                            acc.at[pl.ds(NS * _ZR, _ZTAIL)])
        plsc.subcore_barrier()

        def step(i, carry):
            eoff = sid * _EPT + i * _SB
            pltpu.sync_copy(dst_hbm.at[pl.ds(eoff, _SB)], idxb)
            pltpu.sync_copy(msgf_hbm.at[pl.ds(chunk * E + eoff, _SB)], vbuf)
            pltpu.sync_copy(vbuf, acc.at[idxb], add=True)
            return carry

        lax.fori_loop(0, _EPT // _SB, step, 0)
        plsc.subcore_barrier()
        for j in range(_ZR // _ZB):
            pltpu.sync_copy(acc.at[pl.ds(nbase + j * _ZB, _ZB)], zbuf)
            pltpu.sync_copy(
                zbuf, unf_hbm.at[pl.ds(chunk * N + nbase + j * _ZB, _ZB)])

        @pl.when(sid == NS - 1)
        def _():
            pltpu.sync_copy(acc.at[pl.ds(NS * _ZR, _ZTAIL)],
                            zbuf.at[pl.ds(0, _ZTAIL)])
            pltpu.sync_copy(zbuf.at[pl.ds(0, _ZTAIL)],
                            unf_hbm.at[pl.ds(chunk * N + NS * _ZR, _ZTAIL)])
        plsc.subcore_barrier()
        pltpu.sync_copy(z128_hbm.at[pl.ds(0, _ZB)], zbuf)

    for j in range(_ZR // _ZB):
        pltpu.sync_copy(zbuf, acc.at[pl.ds(nbase + j * _ZB, _ZB)])

    @pl.when(sid == NS - 1)
    def _():
        pltpu.sync_copy(zbuf.at[pl.ds(0, _ZTAIL)],
                        acc.at[pl.ds(NS * _ZR, _ZTAIL)])
    plsc.subcore_barrier()

    def step4(i, carry):
        eoff = cid * (E // 2) + sid * (_EPT // 2) + i * _SB4
        pltpu.sync_copy(dst_hbm.at[pl.ds(eoff, _SB4)], idxb4)
        pltpu.sync_copy(msgf_hbm.at[pl.ds(4 * E + eoff, _SB4)], vbuf4)
        pltpu.sync_copy(vbuf4, acc.at[idxb4], add=True)
        return carry

    lax.fori_loop(0, (_EPT // 2) // _SB4, step4, 0)
    plsc.subcore_barrier()
    for j in range(_ZR // _ZB):
        pltpu.sync_copy(acc.at[pl.ds(nbase + j * _ZB, _ZB)], zbuf)
        pltpu.sync_copy(
            zbuf, unf_hbm.at[pl.ds((4 + cid) * N + nbase + j * _ZB, _ZB)])

    @pl.when(sid == NS - 1)
    def _():
        pltpu.sync_copy(acc.at[pl.ds(NS * _ZR, _ZTAIL)],
                        zbuf.at[pl.ds(0, _ZTAIL)])
        pltpu.sync_copy(zbuf.at[pl.ds(0, _ZTAIL)],
                        unf_hbm.at[pl.ds((4 + cid) * N + NS * _ZR, _ZTAIL)])


@functools.cache
def _scatter():
    return pl.kernel(
        _scatter_body,
        out_type=jax.ShapeDtypeStruct((6 * N, 128), jnp.float32),
        mesh=_sc_mesh(),
        scratch_types=[
            pltpu.VMEM_SHARED((N, 128), jnp.float32),
            pltpu.VMEM((_SB, 128), jnp.float32),
            pltpu.VMEM((_SB,), jnp.int32),
            pltpu.VMEM((_ZB, 128), jnp.float32),
            pltpu.VMEM((_SB4, 128), jnp.float32),
            pltpu.VMEM((_SB4,), jnp.int32),
        ],
    )

_NB = 2000


def _norm_body(u_ref, x_ref, b_ref, o_ref):
    d = u_ref[4] + u_ref[5]
    inv = 1.0 / (d + 1e-16)
    scale = jnp.dot(inv, x_ref[...], preferred_element_type=jnp.float32)
    u = jnp.concatenate([u_ref[c] for c in range(4)], axis=1)
    o_ref[...] = u * scale + b_ref[...]


_norm = pl.pallas_call(
    _norm_body,
    grid=(N // _NB,),
    in_specs=[
        pl.BlockSpec((6, _NB, 128), lambda i: (0, i, 0)),
        pl.BlockSpec((128, F), lambda i: (0, 0)),
        pl.BlockSpec((1, F), lambda i: (0, 0)),
    ],
    out_specs=pl.BlockSpec((_NB, F), lambda i: (i, 0)),
    out_shape=jax.ShapeDtypeStruct((N, F), jnp.float32),
)


def kernel(feats, edge_index, W_src, W_dst, attn, bias):
    fs, fd = _mm(feats, W_src, W_dst)
    src = edge_index[0]
    dst = edge_index[1]
    ga, gb = _gather()(fs, fd, src, dst)

    aflat = attn.reshape(F)
    head = jnp.arange(F, dtype=jnp.int32) // OUT_FEATS
    sel = head[:, None] == jnp.arange(16, dtype=jnp.int32)[None, :]
    a16 = jnp.where(sel, aflat[:, None], 0.0)
    x16 = sel.T.astype(jnp.float32)

    msg5 = _edge(ga, gb, a16, x16)

    z128 = jnp.zeros((_ZR, 128), jnp.float32)
    unf = _scatter()(msg5.reshape(5 * E, 128), dst, z128)

    x128 = jnp.concatenate([x16, jnp.zeros((112, F), jnp.float32)], axis=0)
    return _norm(unf.reshape(6, N, 128), x128, bias.reshape(1, F))


# speedup vs baseline: 18.3901x; 1.4882x over previous
import functools

import jax
import jax.numpy as jnp
from jax import lax
from jax.experimental import pallas as pl
from jax.experimental.pallas import tpu as pltpu
from jax.experimental.pallas import tpu_sc as plsc

N = 10000
E = 160000
IN_FEATS = 256
OUT_FEATS = 64
HEADS = 8
F = HEADS * OUT_FEATS
NEG_SLOPE = 0.2

NC = 2
NS = 16
NW = NC * NS

_MB = 1000


def _mm_body(x_ref, ws_ref, wd_ref, fs_ref, fd_ref):
    x = x_ref[...]
    fs_ref[...] = jnp.dot(x, ws_ref[...], preferred_element_type=jnp.float32)
    fd_ref[...] = jnp.dot(x, wd_ref[...], preferred_element_type=jnp.float32)


_mm = pl.pallas_call(
    _mm_body,
    grid=(N // _MB,),
    in_specs=[
        pl.BlockSpec((_MB, IN_FEATS), lambda i: (i, 0)),
        pl.BlockSpec((IN_FEATS, F), lambda i: (0, 0)),
        pl.BlockSpec((IN_FEATS, F), lambda i: (0, 0)),
    ],
    out_specs=[
        pl.BlockSpec((_MB, F), lambda i: (i, 0)),
        pl.BlockSpec((_MB, F), lambda i: (i, 0)),
    ],
    out_shape=[jax.ShapeDtypeStruct((N, F), jnp.float32)] * 2,
)

_GB = 40
_EPW = E // NW

@functools.cache
def _sc_mesh():
    return plsc.VectorSubcoreMesh(
        core_axis_name="c", subcore_axis_name="s", num_cores=NC, num_subcores=NS)


_GNI = _EPW // _GB


def _gather_body(fs_hbm, fd_hbm, srcm_hbm, dstm_hbm, ga_hbm, gb_hbm,
                 sim, dim, abuf0, abuf1, bbuf0, bbuf1,
                 sga0, sga1, sgb0, sgb1, swa0, swa1, swb0, swb1):
    wid = lax.axis_index("s") * NC + lax.axis_index("c")
    base = wid * _EPW
    abuf = (abuf0, abuf1)
    bbuf = (bbuf0, bbuf1)
    sga = (sga0, sga1)
    sgb = (sgb0, sgb1)
    swa = (swa0, swa1)
    swb = (swb0, swb1)

    pltpu.sync_copy(srcm_hbm.at[wid], sim)
    pltpu.sync_copy(dstm_hbm.at[wid], dim)

    def g_issue(i, b):
        pltpu.async_copy(fs_hbm.at[sim.at[i]], abuf[b], sga[b])
        pltpu.async_copy(fd_hbm.at[dim.at[i]], bbuf[b], sgb[b])

    def g_wait(b):
        pltpu.make_async_copy(fs_hbm.at[sim.at[0]], abuf[b], sga[b]).wait()
        pltpu.make_async_copy(fd_hbm.at[dim.at[0]], bbuf[b], sgb[b]).wait()

    def w_issue(i, b):
        off = base + i * _GB
        pltpu.async_copy(abuf[b], ga_hbm.at[pl.ds(off, _GB)], swa[b])
        pltpu.async_copy(bbuf[b], gb_hbm.at[pl.ds(off, _GB)], swb[b])

    def w_wait(b):
        pltpu.make_async_copy(abuf[b], ga_hbm.at[pl.ds(0, _GB)], swa[b]).wait()
        pltpu.make_async_copy(bbuf[b], gb_hbm.at[pl.ds(0, _GB)], swb[b]).wait()

    g_issue(0, 0)
    g_issue(1, 1)

    def pair(k, carry):
        for b in range(2):
            i = 2 * k + b
            g_wait(b)
            w_issue(i, b)

            @pl.when(i + 2 <= _GNI - 1)
            def _():
                w_wait(b)
                g_issue(i + 2, b)
        return carry

    lax.fori_loop(0, (_GNI - 1) // 2, pair, 0)
    g_wait(0)
    w_issue(_GNI - 1, 0)
    w_wait(0)
    w_wait(1)


@functools.cache
def _gather():
    return pl.kernel(
        _gather_body,
        out_type=[jax.ShapeDtypeStruct((E, F), jnp.float32),
                  jax.ShapeDtypeStruct((E, F), jnp.float32)],
        mesh=_sc_mesh(),
        scratch_types=[
            pltpu.VMEM((_GNI, _GB), jnp.int32),
            pltpu.VMEM((_GNI, _GB), jnp.int32),
            pltpu.VMEM((_GB, F), jnp.float32),
            pltpu.VMEM((_GB, F), jnp.float32),
            pltpu.VMEM((_GB, F), jnp.float32),
            pltpu.VMEM((_GB, F), jnp.float32),
        ] + [pltpu.SemaphoreType.DMA] * 8,
    )

_EB = 2000


def _edge_body(ga_ref, gb_ref, a_ref, x_ref, msg_ref):
    ga = ga_ref[...]
    t = ga + gb_ref[...]
    t = jnp.maximum(t, NEG_SLOPE * t)
    ex = jnp.exp(jnp.dot(t, a_ref[...], preferred_element_type=jnp.float32))
    msg = ga * jnp.dot(ex, x_ref[...], preferred_element_type=jnp.float32)
    for c in range(4):
        msg_ref[c] = msg[:, c * 128:(c + 1) * 128]
    msg_ref[4] = jnp.concatenate(
        [ex, jnp.zeros((_EB, 112), jnp.float32)], axis=1)


_edge = pl.pallas_call(
    _edge_body,
    grid=(E // _EB,),
    in_specs=[
        pl.BlockSpec((_EB, F), lambda i: (i, 0)),
        pl.BlockSpec((_EB, F), lambda i: (i, 0)),
        pl.BlockSpec((F, 16), lambda i: (0, 0)),
        pl.BlockSpec((16, F), lambda i: (0, 0)),
    ],
    out_specs=pl.BlockSpec((5, _EB, 128), lambda i: (0, i, 0)),
    out_shape=jax.ShapeDtypeStruct((5, E, 128), jnp.float32),
)

_SB = 80
_EPT = E // NS
_ZR = 624
_ZTAIL = N - NS * _ZR


_ZB = 48


_SB4 = 40
_SNI = _EPT // _SB
_SNI4 = (_EPT // 2) // _SB4


def _scatter_body(msgf_hbm, dst_hbm, z128_hbm,
                  unf_hbm, acc, vb0, vb1, ib0, ib1, i40, i41,
                  zb0, zb1, sr0, sr1, sz, sd0, sd1):
    cid = lax.axis_index("c")
    sid = lax.axis_index("s")
    nbase = sid * _ZR
    vb = (vb0, vb1)
    ib = (ib0, ib1)
    ib4 = (i40, i41)
    zb = (zb0, zb1)
    sr = (sr0, sr1)
    sd = (sd0, sd1)

    pltpu.sync_copy(z128_hbm.at[pl.ds(0, _ZB)], zb0)

    def zero_acc():
        for j in range(_ZR // _ZB):
            pltpu.async_copy(zb0, acc.at[pl.ds(nbase + j * _ZB, _ZB)], sz)
        for j in range(_ZR // _ZB):
            pltpu.make_async_copy(zb0, acc.at[pl.ds(nbase, _ZB)], sz).wait()

        @pl.when(sid == NS - 1)
        def _():
            pltpu.sync_copy(zb0.at[pl.ds(0, _ZTAIL)],
                            acc.at[pl.ds(NS * _ZR, _ZTAIL)])

    def drain(plane):
        for j in range(_ZR // _ZB):
            b = j % 2
            if j >= 2:
                pltpu.make_async_copy(
                    zb[b], unf_hbm.at[pl.ds(0, _ZB)], sd[b]).wait()
            pltpu.sync_copy(acc.at[pl.ds(nbase + j * _ZB, _ZB)], zb[b])
            pltpu.async_copy(
                zb[b], unf_hbm.at[pl.ds(plane * N + nbase + j * _ZB, _ZB)],
                sd[b])
        for b in range(2):
            pltpu.make_async_copy(
                zb[b], unf_hbm.at[pl.ds(0, _ZB)], sd[b]).wait()

        @pl.when(sid == NS - 1)
        def _():
            pltpu.sync_copy(acc.at[pl.ds(NS * _ZR, _ZTAIL)],
                            zb0.at[pl.ds(0, _ZTAIL)])
            pltpu.sync_copy(zb0.at[pl.ds(0, _ZTAIL)],
                            unf_hbm.at[pl.ds(plane * N + NS * _ZR, _ZTAIL)])
        pltpu.sync_copy(z128_hbm.at[pl.ds(0, _ZB)], zb0)

    for rep in range(2):
        chunk = cid * 2 + rep
        ebase = chunk * E + sid * _EPT
        zero_acc()
        plsc.subcore_barrier()

        def r_issue(i, b):
            eoff = sid * _EPT + i * _SB
            pltpu.async_copy(dst_hbm.at[pl.ds(eoff, _SB)], ib[b], sr[b])
            pltpu.async_copy(
                msgf_hbm.at[pl.ds(ebase + i * _SB, _SB)], vb[b], sr[b])

        def r_wait(b):
            pltpu.make_async_copy(dst_hbm.at[pl.ds(0, _SB)], ib[b],
                                  sr[b]).wait()
            pltpu.make_async_copy(
                msgf_hbm.at[pl.ds(0, _SB)], vb[b], sr[b]).wait()

        r_issue(0, 0)
        r_issue(1, 1)

        def pair(k, carry):
            for b in range(2):
                i = 2 * k + b
                r_wait(b)
                pltpu.sync_copy(vb[b], acc.at[ib[b]], add=True)

                @pl.when(i + 2 <= _SNI - 1)
                def _():
                    r_issue(i + 2, b)
            return carry

        lax.fori_loop(0, (_SNI - 1) // 2, pair, 0)
        r_wait(0)
        pltpu.sync_copy(vb[0], acc.at[ib[0]], add=True)
        plsc.subcore_barrier()
        drain(chunk)
        plsc.subcore_barrier()

    zero_acc()
    plsc.subcore_barrier()
    ebase4 = 4 * E + cid * (E // 2) + sid * (_EPT // 2)

    eibase4 = cid * (E // 2) + sid * (_EPT // 2)

    def r4_issue(i, b):
        pltpu.async_copy(dst_hbm.at[pl.ds(eibase4 + i * _SB4, _SB4)],
                         ib4[b], sr[b])
        pltpu.async_copy(
            msgf_hbm.at[pl.ds(ebase4 + i * _SB4, _SB4)],
            vb[b].at[pl.ds(0, _SB4)], sr[b])

    def r4_wait(b):
        pltpu.make_async_copy(dst_hbm.at[pl.ds(0, _SB4)], ib4[b],
                              sr[b]).wait()
        pltpu.make_async_copy(
            msgf_hbm.at[pl.ds(0, _SB4)], vb[b].at[pl.ds(0, _SB4)],
            sr[b]).wait()

    r4_issue(0, 0)
    r4_issue(1, 1)

    def pair4(k, carry):
        for b in range(2):
            i = 2 * k + b
            r4_wait(b)
            pltpu.sync_copy(vb[b].at[pl.ds(0, _SB4)], acc.at[ib4[b]],
                            add=True)

            @pl.when(i + 2 <= _SNI4 - 1)
            def _():
                r4_issue(i + 2, b)
        return carry

    lax.fori_loop(0, (_SNI4 - 1) // 2, pair4, 0)
    r4_wait(0)
    pltpu.sync_copy(vb[0].at[pl.ds(0, _SB4)], acc.at[ib4[0]], add=True)
    plsc.subcore_barrier()
    drain(4 + cid)


@functools.cache
def _scatter():
    return pl.kernel(
        _scatter_body,
        out_type=jax.ShapeDtypeStruct((6 * N, 128), jnp.float32),
        mesh=_sc_mesh(),
        scratch_types=[
            pltpu.VMEM_SHARED((N, 128), jnp.float32),
            pltpu.VMEM((_SB, 128), jnp.float32),
            pltpu.VMEM((_SB, 128), jnp.float32),
            pltpu.VMEM((_SB,), jnp.int32),
            pltpu.VMEM((_SB,), jnp.int32),
            pltpu.VMEM((_SB4,), jnp.int32),
            pltpu.VMEM((_SB4,), jnp.int32),
            pltpu.VMEM((_ZB, 128), jnp.float32),
            pltpu.VMEM((_ZB, 128), jnp.float32),
        ] + [pltpu.SemaphoreType.DMA] * 5,
    )

_NB = 2000


def _norm_body(u_ref, x_ref, b_ref, o_ref):
    d = u_ref[4] + u_ref[5]
    inv = 1.0 / (d + 1e-16)
    scale = jnp.dot(inv, x_ref[...], preferred_element_type=jnp.float32)
    u = jnp.concatenate([u_ref[c] for c in range(4)], axis=1)
    o_ref[...] = u * scale + b_ref[...]


_norm = pl.pallas_call(
    _norm_body,
    grid=(N // _NB,),
    in_specs=[
        pl.BlockSpec((6, _NB, 128), lambda i: (0, i, 0)),
        pl.BlockSpec((128, F), lambda i: (0, 0)),
        pl.BlockSpec((1, F), lambda i: (0, 0)),
    ],
    out_specs=pl.BlockSpec((_NB, F), lambda i: (i, 0)),
    out_shape=jax.ShapeDtypeStruct((N, F), jnp.float32),
)


def kernel(feats, edge_index, W_src, W_dst, attn, bias):
    fs, fd = _mm(feats, W_src, W_dst)
    src = edge_index[0]
    dst = edge_index[1]
    ga, gb = _gather()(fs, fd, src.reshape(NW, _GNI, _GB),
                       dst.reshape(NW, _GNI, _GB))

    aflat = attn.reshape(F)
    head = jnp.arange(F, dtype=jnp.int32) // OUT_FEATS
    sel = head[:, None] == jnp.arange(16, dtype=jnp.int32)[None, :]
    a16 = jnp.where(sel, aflat[:, None], 0.0)
    x16 = sel.T.astype(jnp.float32)

    msg5 = _edge(ga, gb, a16, x16)

    z128 = jnp.zeros((_ZR, 128), jnp.float32)
    unf = _scatter()(msg5.reshape(5 * E, 128), dst, z128)

    x128 = jnp.concatenate([x16, jnp.zeros((112, F), jnp.float32)], axis=0)
    return _norm(unf.reshape(6, N, 128), x128, bias.reshape(1, F))
